# baseline (speedup 1.0000x reference)
import functools

import jax
import jax.numpy as jnp
from jax import lax
from jax.experimental import pallas as pl
from jax.experimental.pallas import tpu as pltpu
from jax.experimental.pallas import tpu_sc as plsc

_IN_CH = 2048
_OUT_CH = 4096
_K = 50
_DECAY = 0.95
_BATCH = 64

_NBLK = 2
_BLK = _OUT_CH // _NBLK

_NT = 16
_CPT = _OUT_CH // _NT
_VPT = _CPT // 16


def _matmul_body(x_ref, w_ref, out_ref):
    out_ref[...] = jax.lax.dot_general(
        x_ref[...], w_ref[...],
        dimension_numbers=(((1,), (1,)), ((), ())),
        preferred_element_type=jnp.float32,
    )


def _kv_sort(kv):
    return plsc.sort_key_val(kv[0], kv[1], descending=True)


def _rev(kv):
    return (lax.rev(kv[0], (0,)), lax.rev(kv[1], (0,)))


def _exchange(a, b):
    pred = (a[0] > b[0]) | ((a[0] == b[0]) & (a[1] < b[1]))
    hi = (jnp.where(pred, a[0], b[0]), jnp.where(pred, a[1], b[1]))
    lo = (jnp.where(pred, b[0], a[0]), jnp.where(pred, b[1], a[1]))
    return hi, lo


def _take_hi(a, b):
    pred = (a[0] > b[0]) | ((a[0] == b[0]) & (a[1] < b[1]))
    return (jnp.where(pred, a[0], b[0]), jnp.where(pred, a[1], b[1]))


def _merge_16_16_full(a, b):
    hi, lo = _exchange(a, _rev(b))
    return [_kv_sort(hi), _kv_sort(lo)]


def _merge_32_32_full(a, b):
    h0, l0 = _exchange(a[0], _rev(b[1]))
    h1, l1 = _exchange(a[1], _rev(b[0]))
    hh, hl = _exchange(h0, h1)
    lh, ll = _exchange(l0, l1)
    return [_kv_sort(hh), _kv_sort(hl), _kv_sort(lh), _kv_sort(ll)]


def _merge_64_64_top(a, b):
    t = [_take_hi(a[i], _rev(b[3 - i])) for i in range(4)]
    h0, l0 = _exchange(t[0], t[2])
    h1, l1 = _exchange(t[1], t[3])
    hh, hl = _exchange(h0, h1)
    lh, ll = _exchange(l0, l1)
    return [_kv_sort(hh), _kv_sort(hl), _kv_sort(lh), _kv_sort(ll)]


def _local_top64(leaves):
    s32 = [_merge_16_16_full(leaves[t], leaves[t + 1]) for t in range(0, 16, 2)]
    s64 = [_merge_32_32_full(s32[t], s32[t + 1]) for t in range(0, 8, 2)]
    t64 = [_merge_64_64_top(s64[t], s64[t + 1]) for t in range(0, 4, 2)]
    return _merge_64_64_top(t64[0], t64[1])


def _sc_filter(enc):
    mesh = plsc.VectorSubcoreMesh(
        core_axis_name="c", subcore_axis_name="s", num_cores=1)

    @functools.partial(
        pl.kernel,
        mesh=mesh,
        out_type=jax.ShapeDtypeStruct((_BATCH, _OUT_CH), jnp.float32),
        compiler_params=pltpu.CompilerParams(
            needs_layout_passes=False, use_tc_tiling_on_sc=False),
        scratch_types=[
            pltpu.VMEM((_BATCH, _CPT), jnp.float32),
            pltpu.VMEM((_BATCH, _CPT), jnp.float32),
            pltpu.VMEM((_CPT,), jnp.float32),
            pltpu.VMEM((_CPT,), jnp.float32),
            pltpu.VMEM((128,), jnp.float32),
            pltpu.VMEM((_NT, 128), jnp.float32),
            pltpu.VMEM_SHARED((_NT, 128), jnp.float32),
        ],
    )
    def filt(enc_hbm, out_hbm, enc_v, out_v, fired_v, inh_v, pub, allp, shp):
        sid = lax.axis_index("s")
        base = sid * _CPT
        pltpu.sync_copy(enc_hbm.at[:, pl.ds(base, _CPT)], enc_v)

        zeros16 = jnp.zeros((16,), jnp.float32)
        ones16 = jnp.ones((16,), jnp.float32)
        lane = lax.iota(jnp.int32, 16)

        for j in range(_VPT):
            inh_v[pl.ds(j * 16, 16)] = zeros16

        def step(i, carry):
            leaves = []
            for j in range(_VPT):
                e = enc_v[i, pl.ds(j * 16, 16)]
                inh = inh_v[pl.ds(j * 16, 16)]
                kk = jnp.abs(e) * (1.0 - inh)
                vv = lane + (base + j * 16)
                leaves.append(_kv_sort((kk, vv)))
                fired_v[pl.ds(j * 16, 16)] = zeros16

            lk = _local_top64(leaves)
            for j in range(4):
                pub[pl.ds(j * 16, 16)] = lk[j][0]
                pub[pl.ds(64 + j * 16, 16)] = plsc.bitcast(lk[j][1], jnp.float32)
            pltpu.sync_copy(pub, shp.at[sid])
            plsc.subcore_barrier()
            pltpu.sync_copy(shp, allp)
            plsc.subcore_barrier()

            tops = []
            for t in range(_NT):
                tops.append([
                    (allp[t, pl.ds(j * 16, 16)],
                     plsc.bitcast(allp[t, pl.ds(64 + j * 16, 16)], jnp.int32))
                    for j in range(4)
                ])
            while len(tops) > 1:
                tops = [_merge_64_64_top(tops[t], tops[t + 1])
                        for t in range(0, len(tops), 2)]
            g = tops[0]

            for j, take in enumerate((16, 16, 16, _K - 48)):
                local = g[j][1] - base
                own = (lane < take) & (local >= 0) & (local < _CPT)
                local_c = jnp.minimum(jnp.maximum(local, 0), _CPT - 1)
                plsc.store_scatter(fired_v, [local_c], ones16, mask=own)

            for j in range(_VPT):
                f = fired_v[pl.ds(j * 16, 16)]
                e = enc_v[i, pl.ds(j * 16, 16)]
                out_v[i, pl.ds(j * 16, 16)] = e * f
                inh = inh_v[pl.ds(j * 16, 16)]
                inh_v[pl.ds(j * 16, 16)] = inh * _DECAY + f
            return carry

        lax.fori_loop(0, _BATCH, step, 0)
        pltpu.sync_copy(out_v, out_hbm.at[:, pl.ds(base, _CPT)])

    return filt(enc)


@jax.jit
def kernel(inputs, W):
    x = inputs.reshape(_BATCH, _IN_CH)
    enc = pl.pallas_call(
        _matmul_body,
        grid=(_NBLK,),
        in_specs=[
            pl.BlockSpec((_BATCH, _IN_CH), lambda j: (0, 0)),
            pl.BlockSpec((_BLK, _IN_CH), lambda j: (j, 0)),
        ],
        out_specs=pl.BlockSpec((_BATCH, _BLK), lambda j: (0, j)),
        out_shape=jax.ShapeDtypeStruct((_BATCH, _OUT_CH), jnp.float32),
    )(x, W)

    return _sc_filter(enc)


# speedup vs baseline: 1.0102x; 1.0102x over previous
import functools

import jax
import jax.numpy as jnp
from jax.experimental.layout import Format, Layout
from jax import lax
from jax.experimental import pallas as pl
from jax.experimental.pallas import tpu as pltpu
from jax.experimental.pallas import tpu_sc as plsc

_IN_CH = 2048
_OUT_CH = 4096
_K = 50
_DECAY = 0.95
_BATCH = 64

_NBLK = 4
_BLK = _OUT_CH // _NBLK

_NT = 16
_CPT = _OUT_CH // _NT
_VPT = _CPT // 16


def _matmul_body(x_ref, w_ref, out_ref):
    out_ref[...] = jax.lax.dot_general(
        x_ref[...], w_ref[...],
        dimension_numbers=(((1,), (1,)), ((), ())),
        preferred_element_type=jnp.float32,
    )


def _kv_sort(kv):
    return plsc.sort_key_val(kv[0], kv[1], descending=True)


def _rev(kv):
    return (lax.rev(kv[0], (0,)), lax.rev(kv[1], (0,)))


def _exchange(a, b):
    pred = (a[0] > b[0]) | ((a[0] == b[0]) & (a[1] < b[1]))
    hi = (jnp.where(pred, a[0], b[0]), jnp.where(pred, a[1], b[1]))
    lo = (jnp.where(pred, b[0], a[0]), jnp.where(pred, b[1], a[1]))
    return hi, lo


def _take_hi(a, b):
    pred = (a[0] > b[0]) | ((a[0] == b[0]) & (a[1] < b[1]))
    return (jnp.where(pred, a[0], b[0]), jnp.where(pred, a[1], b[1]))


def _merge_16_16_full(a, b):
    hi, lo = _exchange(a, _rev(b))
    return [_kv_sort(hi), _kv_sort(lo)]


def _merge_32_32_full(a, b):
    h0, l0 = _exchange(a[0], _rev(b[1]))
    h1, l1 = _exchange(a[1], _rev(b[0]))
    hh, hl = _exchange(h0, h1)
    lh, ll = _exchange(l0, l1)
    return [_kv_sort(hh), _kv_sort(hl), _kv_sort(lh), _kv_sort(ll)]


def _merge_64_64_top(a, b):
    t = [_take_hi(a[i], _rev(b[3 - i])) for i in range(4)]
    h0, l0 = _exchange(t[0], t[2])
    h1, l1 = _exchange(t[1], t[3])
    hh, hl = _exchange(h0, h1)
    lh, ll = _exchange(l0, l1)
    return [_kv_sort(hh), _kv_sort(hl), _kv_sort(lh), _kv_sort(ll)]


def _local_top64(leaves):
    s32 = [_merge_16_16_full(leaves[t], leaves[t + 1]) for t in range(0, 16, 2)]
    s64 = [_merge_32_32_full(s32[t], s32[t + 1]) for t in range(0, 8, 2)]
    t64 = [_merge_64_64_top(s64[t], s64[t + 1]) for t in range(0, 4, 2)]
    return _merge_64_64_top(t64[0], t64[1])


def _sc_filter(enc):
    mesh = plsc.VectorSubcoreMesh(
        core_axis_name="c", subcore_axis_name="s", num_cores=1)

    @functools.partial(
        pl.kernel,
        mesh=mesh,
        out_type=jax.ShapeDtypeStruct((_BATCH, _OUT_CH), jnp.float32),
        compiler_params=pltpu.CompilerParams(
            needs_layout_passes=False, use_tc_tiling_on_sc=False),
        scratch_types=[
            pltpu.VMEM((_BATCH, _CPT), jnp.float32),
            pltpu.VMEM((_BATCH, _CPT), jnp.float32),
            pltpu.VMEM((_CPT,), jnp.float32),
            pltpu.VMEM((_CPT,), jnp.float32),
            pltpu.VMEM((128,), jnp.float32),
            pltpu.VMEM((_NT, 128), jnp.float32),
            pltpu.VMEM_SHARED((_NT, 128), jnp.float32),
        ],
    )
    def filt(enc_hbm, out_hbm, enc_v, out_v, fired_v, inh_v, pub, allp, shp):
        sid = lax.axis_index("s")
        base = sid * _CPT
        pltpu.sync_copy(enc_hbm.at[:, pl.ds(base, _CPT)], enc_v)

        zeros16 = jnp.zeros((16,), jnp.float32)
        ones16 = jnp.ones((16,), jnp.float32)
        lane = lax.iota(jnp.int32, 16)

        for j in range(_VPT):
            inh_v[pl.ds(j * 16, 16)] = zeros16

        def step(i, carry):
            leaves = []
            for j in range(_VPT):
                e = enc_v[i, pl.ds(j * 16, 16)]
                inh = inh_v[pl.ds(j * 16, 16)]
                kk = jnp.abs(e) * (1.0 - inh)
                vv = lane + (base + j * 16)
                leaves.append(_kv_sort((kk, vv)))
                fired_v[pl.ds(j * 16, 16)] = zeros16

            lk = _local_top64(leaves)
            for j in range(4):
                pub[pl.ds(j * 16, 16)] = lk[j][0]
                pub[pl.ds(64 + j * 16, 16)] = plsc.bitcast(lk[j][1], jnp.float32)
            pltpu.sync_copy(pub, shp.at[sid])
            plsc.subcore_barrier()
            pltpu.sync_copy(shp, allp)
            plsc.subcore_barrier()

            tops = []
            for t in range(_NT):
                tops.append([
                    (allp[t, pl.ds(j * 16, 16)],
                     plsc.bitcast(allp[t, pl.ds(64 + j * 16, 16)], jnp.int32))
                    for j in range(4)
                ])
            while len(tops) > 1:
                tops = [_merge_64_64_top(tops[t], tops[t + 1])
                        for t in range(0, len(tops), 2)]
            g = tops[0]

            for j, take in enumerate((16, 16, 16, _K - 48)):
                local = g[j][1] - base
                own = (lane < take) & (local >= 0) & (local < _CPT)
                local_c = jnp.minimum(jnp.maximum(local, 0), _CPT - 1)
                plsc.store_scatter(fired_v, [local_c], ones16, mask=own)

            for j in range(_VPT):
                f = fired_v[pl.ds(j * 16, 16)]
                e = enc_v[i, pl.ds(j * 16, 16)]
                out_v[i, pl.ds(j * 16, 16)] = e * f
                inh = inh_v[pl.ds(j * 16, 16)]
                inh_v[pl.ds(j * 16, 16)] = inh * _DECAY + f
            return carry

        lax.fori_loop(0, _BATCH, step, 0)
        pltpu.sync_copy(out_v, out_hbm.at[:, pl.ds(base, _CPT)])

    return filt(enc)


def _kernel_impl(inputs, W):
    x = inputs.reshape(_BATCH, _IN_CH)
    enc = pl.pallas_call(
        _matmul_body,
        grid=(_NBLK,),
        in_specs=[
            pl.BlockSpec((_BATCH, _IN_CH), lambda j: (0, 0)),
            pl.BlockSpec((_BLK, _IN_CH), lambda j: (j, 0)),
        ],
        out_specs=pl.BlockSpec((_BATCH, _BLK), lambda j: (0, j)),
        out_shape=jax.ShapeDtypeStruct((_BATCH, _OUT_CH), jnp.float32),
    )(x, W)

    return _sc_filter(enc)


_jitted = None


def kernel(inputs, W):
    global _jitted
    if _jitted is None:
        fmt = Format(
            Layout(major_to_minor=(0, 1), tiling=()),
            jax.sharding.SingleDeviceSharding(jax.devices()[0]),
        )
        _jitted = jax.jit(_kernel_impl, out_shardings=fmt)
    return _jitted(inputs, W)


# speedup vs baseline: 1.0129x; 1.0027x over previous
import functools

import jax
import jax.numpy as jnp
from jax import lax
from jax.experimental import pallas as pl
from jax.experimental.pallas import tpu as pltpu
from jax.experimental.pallas import tpu_sc as plsc

_IN_CH = 2048
_OUT_CH = 4096
_K = 50
_DECAY = 0.95
_BATCH = 64

_NBLK = 4
_BLK = _OUT_CH // _NBLK

_NT = 16
_CPT = _OUT_CH // _NT
_VPT = _CPT // 16


def _matmul_body(x_ref, w_ref, out_ref):
    out_ref[...] = jax.lax.dot_general(
        x_ref[...], w_ref[...],
        dimension_numbers=(((1,), (1,)), ((), ())),
        preferred_element_type=jnp.float32,
    )


def _kv_sort(kv):
    return plsc.sort_key_val(kv[0], kv[1], descending=True)


def _rev(kv):
    return (lax.rev(kv[0], (0,)), lax.rev(kv[1], (0,)))


def _exchange(a, b):
    pred = (a[0] > b[0]) | ((a[0] == b[0]) & (a[1] < b[1]))
    hi = (jnp.where(pred, a[0], b[0]), jnp.where(pred, a[1], b[1]))
    lo = (jnp.where(pred, b[0], a[0]), jnp.where(pred, b[1], a[1]))
    return hi, lo


def _take_hi(a, b):
    pred = (a[0] > b[0]) | ((a[0] == b[0]) & (a[1] < b[1]))
    return (jnp.where(pred, a[0], b[0]), jnp.where(pred, a[1], b[1]))


def _merge_16_16_full(a, b):
    hi, lo = _exchange(a, _rev(b))
    return [_kv_sort(hi), _kv_sort(lo)]


def _merge_32_32_full(a, b):
    h0, l0 = _exchange(a[0], _rev(b[1]))
    h1, l1 = _exchange(a[1], _rev(b[0]))
    hh, hl = _exchange(h0, h1)
    lh, ll = _exchange(l0, l1)
    return [_kv_sort(hh), _kv_sort(hl), _kv_sort(lh), _kv_sort(ll)]


def _merge_64_64_top(a, b):
    t = [_take_hi(a[i], _rev(b[3 - i])) for i in range(4)]
    h0, l0 = _exchange(t[0], t[2])
    h1, l1 = _exchange(t[1], t[3])
    hh, hl = _exchange(h0, h1)
    lh, ll = _exchange(l0, l1)
    return [_kv_sort(hh), _kv_sort(hl), _kv_sort(lh), _kv_sort(ll)]


def _local_top64(leaves):
    s32 = [_merge_16_16_full(leaves[t], leaves[t + 1]) for t in range(0, 16, 2)]
    s64 = [_merge_32_32_full(s32[t], s32[t + 1]) for t in range(0, 8, 2)]
    t64 = [_merge_64_64_top(s64[t], s64[t + 1]) for t in range(0, 4, 2)]
    return _merge_64_64_top(t64[0], t64[1])


def _sc_filter(enc):
    mesh = plsc.VectorSubcoreMesh(
        core_axis_name="c", subcore_axis_name="s", num_cores=1)

    @functools.partial(
        pl.kernel,
        mesh=mesh,
        out_type=jax.ShapeDtypeStruct((_BATCH, _OUT_CH), jnp.float32),
        compiler_params=pltpu.CompilerParams(
            needs_layout_passes=False, use_tc_tiling_on_sc=False),
        scratch_types=[
            pltpu.VMEM((_BATCH, _CPT), jnp.float32),
            pltpu.VMEM((_BATCH, _CPT), jnp.float32),
            pltpu.VMEM((_CPT,), jnp.float32),
            pltpu.VMEM((_CPT,), jnp.float32),
            pltpu.VMEM((128,), jnp.float32),
            pltpu.VMEM((_NT, 128), jnp.float32),
            pltpu.VMEM_SHARED((_NT, 128), jnp.float32),
        ],
    )
    def filt(enc_hbm, out_hbm, enc_v, out_v, fired_v, inh_v, pub, allp, shp):
        sid = lax.axis_index("s")
        base = sid * _CPT
        pltpu.sync_copy(enc_hbm.at[:, pl.ds(base, _CPT)], enc_v)

        zeros16 = jnp.zeros((16,), jnp.float32)
        ones16 = jnp.ones((16,), jnp.float32)
        lane = lax.iota(jnp.int32, 16)

        for j in range(_VPT):
            inh_v[pl.ds(j * 16, 16)] = zeros16

        def step(i, carry):
            leaves = []
            for j in range(_VPT):
                e = enc_v[i, pl.ds(j * 16, 16)]
                inh = inh_v[pl.ds(j * 16, 16)]
                kk = jnp.abs(e) * (1.0 - inh)
                vv = lane + (base + j * 16)
                leaves.append(_kv_sort((kk, vv)))
                fired_v[pl.ds(j * 16, 16)] = zeros16

            lk = _local_top64(leaves)
            for j in range(4):
                pub[pl.ds(j * 16, 16)] = lk[j][0]
                pub[pl.ds(64 + j * 16, 16)] = plsc.bitcast(lk[j][1], jnp.float32)
            pltpu.sync_copy(pub, shp.at[sid])
            plsc.subcore_barrier()
            pltpu.sync_copy(shp, allp)
            plsc.subcore_barrier()

            tops = []
            for t in range(_NT):
                tops.append([
                    (allp[t, pl.ds(j * 16, 16)],
                     plsc.bitcast(allp[t, pl.ds(64 + j * 16, 16)], jnp.int32))
                    for j in range(4)
                ])
            while len(tops) > 1:
                tops = [_merge_64_64_top(tops[t], tops[t + 1])
                        for t in range(0, len(tops), 2)]
            g = tops[0]

            for j, take in enumerate((16, 16, 16, _K - 48)):
                local = g[j][1] - base
                own = (lane < take) & (local >= 0) & (local < _CPT)
                local_c = jnp.minimum(jnp.maximum(local, 0), _CPT - 1)
                plsc.store_scatter(fired_v, [local_c], ones16, mask=own)

            for j in range(_VPT):
                f = fired_v[pl.ds(j * 16, 16)]
                e = enc_v[i, pl.ds(j * 16, 16)]
                out_v[i, pl.ds(j * 16, 16)] = e * f
                inh = inh_v[pl.ds(j * 16, 16)]
                inh_v[pl.ds(j * 16, 16)] = inh * _DECAY + f
            return carry

        lax.fori_loop(0, _BATCH, step, 0)
        pltpu.sync_copy(out_v, out_hbm.at[:, pl.ds(base, _CPT)])

    return filt(enc)


@jax.jit
def kernel(inputs, W):
    x = inputs.reshape(_BATCH, _IN_CH)
    enc = pl.pallas_call(
        _matmul_body,
        grid=(_NBLK,),
        in_specs=[
            pl.BlockSpec((_BATCH, _IN_CH), lambda j: (0, 0)),
            pl.BlockSpec((_BLK, _IN_CH), lambda j: (j, 0)),
        ],
        out_specs=pl.BlockSpec((_BATCH, _BLK), lambda j: (0, j)),
        out_shape=jax.ShapeDtypeStruct((_BATCH, _OUT_CH), jnp.float32),
    )(x, W)

    return _sc_filter(enc)


# speedup vs baseline: 1.0354x; 1.0222x over previous
import functools

import jax
import jax.numpy as jnp
from jax import lax
from jax.experimental import pallas as pl
from jax.experimental.pallas import tpu as pltpu
from jax.experimental.pallas import tpu_sc as plsc

_IN_CH = 2048
_OUT_CH = 4096
_K = 50
_DECAY = 0.95
_BATCH = 64

_NBLK = 4
_BLK = _OUT_CH // _NBLK

_NT = 16
_CPT = _OUT_CH // _NT
_VPT = _CPT // 16


def _matmul_body(x_ref, w_ref, out_ref):
    out_ref[...] = jax.lax.dot_general(
        x_ref[...], w_ref[...],
        dimension_numbers=(((1,), (1,)), ((), ())),
        preferred_element_type=jnp.float32,
    )


def _kv_sort(kv):
    return plsc.sort_key_val(kv[0], kv[1], descending=True)


def _rev(kv):
    return (lax.rev(kv[0], (0,)), lax.rev(kv[1], (0,)))


def _exchange(a, b):
    pred = (a[0] > b[0]) | ((a[0] == b[0]) & (a[1] < b[1]))
    hi = (jnp.where(pred, a[0], b[0]), jnp.where(pred, a[1], b[1]))
    lo = (jnp.where(pred, b[0], a[0]), jnp.where(pred, b[1], a[1]))
    return hi, lo


def _take_hi(a, b):
    pred = (a[0] > b[0]) | ((a[0] == b[0]) & (a[1] < b[1]))
    return (jnp.where(pred, a[0], b[0]), jnp.where(pred, a[1], b[1]))


def _merge_16_16_full(a, b):
    hi, lo = _exchange(a, _rev(b))
    return [_kv_sort(hi), _kv_sort(lo)]


def _merge_32_32_full(a, b):
    h0, l0 = _exchange(a[0], _rev(b[1]))
    h1, l1 = _exchange(a[1], _rev(b[0]))
    hh, hl = _exchange(h0, h1)
    lh, ll = _exchange(l0, l1)
    return [_kv_sort(hh), _kv_sort(hl), _kv_sort(lh), _kv_sort(ll)]


def _merge_64_64_top(a, b):
    t = [_take_hi(a[i], _rev(b[3 - i])) for i in range(4)]
    h0, l0 = _exchange(t[0], t[2])
    h1, l1 = _exchange(t[1], t[3])
    hh, hl = _exchange(h0, h1)
    lh, ll = _exchange(l0, l1)
    return [_kv_sort(hh), _kv_sort(hl), _kv_sort(lh), _kv_sort(ll)]


def _local_top64(leaves):
    s32 = [_merge_16_16_full(leaves[t], leaves[t + 1]) for t in range(0, 16, 2)]
    s64 = [_merge_32_32_full(s32[t], s32[t + 1]) for t in range(0, 8, 2)]
    t64 = [_merge_64_64_top(s64[t], s64[t + 1]) for t in range(0, 4, 2)]
    return _merge_64_64_top(t64[0], t64[1])


def _sc_filter(enc):
    mesh = plsc.VectorSubcoreMesh(
        core_axis_name="c", subcore_axis_name="s", num_cores=1)

    @functools.partial(
        pl.kernel,
        mesh=mesh,
        out_type=jax.ShapeDtypeStruct((_BATCH, _OUT_CH), jnp.float32),
        compiler_params=pltpu.CompilerParams(
            needs_layout_passes=False, use_tc_tiling_on_sc=False),
        scratch_types=[
            pltpu.VMEM((_BATCH, _CPT), jnp.float32),
            pltpu.VMEM((_BATCH, _CPT), jnp.float32),
            pltpu.VMEM((_CPT,), jnp.float32),
            pltpu.VMEM((_CPT,), jnp.float32),
            pltpu.VMEM((128,), jnp.float32),
            pltpu.VMEM((_NT, 128), jnp.float32),
            pltpu.VMEM_SHARED((2, _NT, 128), jnp.float32),
        ],
    )
    def filt(enc_hbm, out_hbm, enc_v, out_v, fired_v, inh_v, pub, allp, shp):
        sid = lax.axis_index("s")
        base = sid * _CPT
        pltpu.sync_copy(enc_hbm.at[:, pl.ds(base, _CPT)], enc_v)

        zeros16 = jnp.zeros((16,), jnp.float32)
        ones16 = jnp.ones((16,), jnp.float32)
        lane = lax.iota(jnp.int32, 16)

        for j in range(_VPT):
            inh_v[pl.ds(j * 16, 16)] = zeros16

        def step(i, carry):
            leaves = []
            for j in range(_VPT):
                e = enc_v[i, pl.ds(j * 16, 16)]
                inh = inh_v[pl.ds(j * 16, 16)]
                kk = jnp.abs(e) * (1.0 - inh)
                vv = lane + (base + j * 16)
                leaves.append(_kv_sort((kk, vv)))
                fired_v[pl.ds(j * 16, 16)] = zeros16

            lk = _local_top64(leaves)
            for j in range(4):
                pub[pl.ds(j * 16, 16)] = lk[j][0]
                pub[pl.ds(64 + j * 16, 16)] = plsc.bitcast(lk[j][1], jnp.float32)
            p = jnp.bitwise_and(i, 1)
            pltpu.sync_copy(pub, shp.at[p, sid])
            plsc.subcore_barrier()
            pltpu.sync_copy(shp.at[p], allp)

            tops = []
            for t in range(_NT):
                tops.append([
                    (allp[t, pl.ds(j * 16, 16)],
                     plsc.bitcast(allp[t, pl.ds(64 + j * 16, 16)], jnp.int32))
                    for j in range(4)
                ])
            while len(tops) > 1:
                tops = [_merge_64_64_top(tops[t], tops[t + 1])
                        for t in range(0, len(tops), 2)]
            g = tops[0]

            for j, take in enumerate((16, 16, 16, _K - 48)):
                local = g[j][1] - base
                own = (lane < take) & (local >= 0) & (local < _CPT)
                local_c = jnp.minimum(jnp.maximum(local, 0), _CPT - 1)
                plsc.store_scatter(fired_v, [local_c], ones16, mask=own)

            for j in range(_VPT):
                f = fired_v[pl.ds(j * 16, 16)]
                e = enc_v[i, pl.ds(j * 16, 16)]
                out_v[i, pl.ds(j * 16, 16)] = e * f
                inh = inh_v[pl.ds(j * 16, 16)]
                inh_v[pl.ds(j * 16, 16)] = inh * _DECAY + f
            return carry

        lax.fori_loop(0, _BATCH, step, 0)
        pltpu.sync_copy(out_v, out_hbm.at[:, pl.ds(base, _CPT)])

    return filt(enc)


@jax.jit
def kernel(inputs, W):
    x = inputs.reshape(_BATCH, _IN_CH)
    enc = pl.pallas_call(
        _matmul_body,
        grid=(_NBLK,),
        in_specs=[
            pl.BlockSpec((_BATCH, _IN_CH), lambda j: (0, 0)),
            pl.BlockSpec((_BLK, _IN_CH), lambda j: (j, 0)),
        ],
        out_specs=pl.BlockSpec((_BATCH, _BLK), lambda j: (0, j)),
        out_shape=jax.ShapeDtypeStruct((_BATCH, _OUT_CH), jnp.float32),
    )(x, W)

    return _sc_filter(enc)
